# Initial kernel scaffold; baseline (speedup 1.0000x reference)
import functools

import jax
import jax.numpy as jnp
from jax import lax
from jax.experimental import pallas as pl
from jax.experimental.pallas import tpu as pltpu
from jax.experimental.pallas import tpu_sc as plsc

N_NODES = 10000
N_EDGES = 320000
D = 128
N_GRAPHS = 64
N_CLASSES = 10

NC = 2
NS = 16
NW = NC * NS
CHUNK = 128
CPT = 79
E_PAD = NW * CPT * CHUNK
NPAD = N_NODES + 16
RPT = NPAD // NS
ROWBLK = 1000
NBLK = N_NODES // ROWBLK


def _mesh():
    return plsc.VectorSubcoreMesh(core_axis_name="c", subcore_axis_name="s")


def _zero_rows(ref, nrows, ncols):
    @pl.loop(0, nrows)
    def _(r):
        @pl.loop(0, ncols // 16)
        def _(j):
            ref[r, pl.ds(j * 16, 16)] = jnp.zeros((16,), jnp.float32)


def _sc_degree(dst3):
    @functools.partial(
        pl.kernel,
        out_type=jax.ShapeDtypeStruct((NC, NPAD, 16), jnp.float32),
        mesh=_mesh(),
        scratch_types=[
            pltpu.VMEM((CPT, CHUNK), jnp.int32),
            pltpu.VMEM((CHUNK, 16), jnp.float32),
            pltpu.VMEM((CHUNK, 16), jnp.float32),
            pltpu.VMEM_SHARED((NPAD, 16), jnp.float32),
        ],
    )
    def k(dst_hbm, out_hbm, dst_v, ones_v, zero_v, acc_sh):
        cid = lax.axis_index("c")
        sid = lax.axis_index("s")
        wid = cid * NS + sid

        @pl.loop(0, CHUNK)
        def _(r):
            ones_v[r, pl.ds(0, 16)] = jnp.ones((16,), jnp.float32)
            zero_v[r, pl.ds(0, 16)] = jnp.zeros((16,), jnp.float32)

        @pl.loop(0, 4)
        def _(t):
            pltpu.sync_copy(zero_v, acc_sh.at[pl.ds(sid * RPT + t * CHUNK, CHUNK)])
        pltpu.sync_copy(zero_v.at[pl.ds(0, RPT - 4 * CHUNK)],
                        acc_sh.at[pl.ds(sid * RPT + 4 * CHUNK, RPT - 4 * CHUNK)])
        plsc.subcore_barrier()

        pltpu.sync_copy(dst_hbm.at[wid], dst_v)

        @pl.loop(0, CPT)
        def _(c):
            pltpu.sync_copy(ones_v, acc_sh.at[dst_v.at[c]], add=True)

        plsc.subcore_barrier()
        pltpu.sync_copy(acc_sh.at[pl.ds(sid * RPT, RPT)],
                        out_hbm.at[cid].at[pl.ds(sid * RPT, RPT)])

    return k(dst3)


def _sc_aggregate(g, src3, dst3):
    @functools.partial(
        pl.kernel,
        out_type=jax.ShapeDtypeStruct((NC, NPAD, D), jnp.float32),
        mesh=_mesh(),
        scratch_types=[
            pltpu.VMEM((CPT, CHUNK), jnp.int32),
            pltpu.VMEM((CPT, CHUNK), jnp.int32),
            pltpu.VMEM((CHUNK, D), jnp.float32),
            pltpu.VMEM((CHUNK, D), jnp.float32),
            pltpu.VMEM_SHARED((NPAD, D), jnp.float32),
            pltpu.SemaphoreType.DMA,
        ],
    )
    def k(g_hbm, src_hbm, dst_hbm, out_hbm, src_v, dst_v, rows_v, zero_v,
          acc_sh, sem):
        cid = lax.axis_index("c")
        sid = lax.axis_index("s")
        wid = cid * NS + sid

        _zero_rows(zero_v, CHUNK, D)

        @pl.loop(0, 4)
        def _(t):
            pltpu.sync_copy(zero_v, acc_sh.at[pl.ds(sid * RPT + t * CHUNK, CHUNK)])
        pltpu.sync_copy(zero_v.at[pl.ds(0, RPT - 4 * CHUNK)],
                        acc_sh.at[pl.ds(sid * RPT + 4 * CHUNK, RPT - 4 * CHUNK)])
        plsc.subcore_barrier()

        pltpu.sync_copy(src_hbm.at[wid], src_v)
        pltpu.sync_copy(dst_hbm.at[wid], dst_v)

        @pl.loop(0, CPT)
        def _(c):
            pltpu.async_copy(g_hbm.at[src_v.at[c]], rows_v, sem).wait()
            pltpu.sync_copy(rows_v, acc_sh.at[dst_v.at[c]], add=True)

        plsc.subcore_barrier()
        pltpu.sync_copy(acc_sh.at[pl.ds(sid * RPT, RPT)],
                        out_hbm.at[cid].at[pl.ds(sid * RPT, RPT)])

    return k(g, src3, dst3)


_DOT = (((1,), (0,)), ((), ()))


def _tc_layer1(x, dega, degb, W1):
    def body(x_ref, da_ref, db_ref, w_ref, g_ref):
        deg = da_ref[:, 0:1] + db_ref[:, 0:1] + 1.0
        dinv = lax.rsqrt(deg)
        h = lax.dot_general(x_ref[...], w_ref[...], _DOT,
                            precision=lax.Precision.HIGHEST)
        g_ref[...] = h * dinv

    return pl.pallas_call(
        body,
        grid=(NBLK,),
        in_specs=[
            pl.BlockSpec((ROWBLK, D), lambda i: (i, 0)),
            pl.BlockSpec((ROWBLK, 16), lambda i: (i, 0)),
            pl.BlockSpec((ROWBLK, 16), lambda i: (i, 0)),
            pl.BlockSpec((D, D), lambda i: (0, 0)),
        ],
        out_specs=pl.BlockSpec((ROWBLK, D), lambda i: (i, 0)),
        out_shape=jax.ShapeDtypeStruct((N_NODES, D), jnp.float32),
    )(x, dega, degb, W1)


def _tc_layer2(acca, accb, g1, dega, degb, b1, W2):
    def body(aa_ref, ab_ref, g_ref, da_ref, db_ref, b_ref, w_ref, o_ref):
        deg = da_ref[:, 0:1] + db_ref[:, 0:1] + 1.0
        dinv = lax.rsqrt(deg)
        z = dinv * (aa_ref[...] + ab_ref[...] + g_ref[...]) + b_ref[...]
        h = jnp.maximum(z, 0.0)
        h2 = lax.dot_general(h, w_ref[...], _DOT,
                             precision=lax.Precision.HIGHEST)
        o_ref[...] = h2 * dinv

    return pl.pallas_call(
        body,
        grid=(NBLK,),
        in_specs=[
            pl.BlockSpec((ROWBLK, D), lambda i: (i, 0)),
            pl.BlockSpec((ROWBLK, D), lambda i: (i, 0)),
            pl.BlockSpec((ROWBLK, D), lambda i: (i, 0)),
            pl.BlockSpec((ROWBLK, 16), lambda i: (i, 0)),
            pl.BlockSpec((ROWBLK, 16), lambda i: (i, 0)),
            pl.BlockSpec((1, D), lambda i: (0, 0)),
            pl.BlockSpec((D, D), lambda i: (0, 0)),
        ],
        out_specs=pl.BlockSpec((ROWBLK, D), lambda i: (i, 0)),
        out_shape=jax.ShapeDtypeStruct((N_NODES, D), jnp.float32),
    )(acca, accb, g1, dega, degb, b1, W2)


def _tc_final(acca, accb, g2, dega, degb, b2, batf, wl, bl):
    def body(aa_ref, ab_ref, g_ref, da_ref, db_ref, b_ref, bat_ref, wl_ref,
             bl_ref, o_ref, pool_ref):
        i = pl.program_id(0)
        deg = da_ref[:, 0:1] + db_ref[:, 0:1] + 1.0
        dinv = lax.rsqrt(deg)
        z = dinv * (aa_ref[...] + ab_ref[...] + g_ref[...]) + b_ref[...]
        h = jnp.maximum(z, 0.0)
        bvec = jnp.reshape(bat_ref[0, 0, :], (1, ROWBLK))
        gids = lax.broadcasted_iota(jnp.int32, (N_GRAPHS, ROWBLK), 0)
        m = (bvec == gids).astype(jnp.float32)
        pm = lax.dot_general(m, h, _DOT, precision=lax.Precision.HIGHEST)

        @pl.when(i == 0)
        def _():
            pool_ref[...] = pm

        @pl.when(i > 0)
        def _():
            pool_ref[...] += pm

        @pl.when(i == NBLK - 1)
        def _():
            o_ref[...] = lax.dot_general(pool_ref[...], wl_ref[...], _DOT,
                                         precision=lax.Precision.HIGHEST) + bl_ref[...]

    return pl.pallas_call(
        body,
        grid=(NBLK,),
        in_specs=[
            pl.BlockSpec((ROWBLK, D), lambda i: (i, 0)),
            pl.BlockSpec((ROWBLK, D), lambda i: (i, 0)),
            pl.BlockSpec((ROWBLK, D), lambda i: (i, 0)),
            pl.BlockSpec((ROWBLK, 16), lambda i: (i, 0)),
            pl.BlockSpec((ROWBLK, 16), lambda i: (i, 0)),
            pl.BlockSpec((1, D), lambda i: (0, 0)),
            pl.BlockSpec((1, 1, ROWBLK), lambda i: (i, 0, 0)),
            pl.BlockSpec((D, D), lambda i: (0, 0)),
            pl.BlockSpec((1, D), lambda i: (0, 0)),
        ],
        out_specs=pl.BlockSpec((N_GRAPHS, D), lambda i: (0, 0)),
        out_shape=jax.ShapeDtypeStruct((N_GRAPHS, D), jnp.float32),
        scratch_shapes=[pltpu.VMEM((N_GRAPHS, D), jnp.float32)],
    )(acca, accb, g2, dega, degb, b2, batf, wl, bl)


def kernel(x, edge_index, batch, W1, b1, W2, b2, Wlin, blin):
    src = edge_index[0].astype(jnp.int32)
    dst = edge_index[1].astype(jnp.int32)
    pad = E_PAD - N_EDGES
    src3 = jnp.concatenate([src, jnp.zeros((pad,), jnp.int32)]).reshape(NW, CPT, CHUNK)
    dst3 = jnp.concatenate([dst, jnp.full((pad,), N_NODES, jnp.int32)]).reshape(NW, CPT, CHUNK)

    deg2 = _sc_degree(dst3)
    dega, degb = deg2[0], deg2[1]

    g1 = _tc_layer1(x, dega, degb, W1)
    acc1 = _sc_aggregate(g1, src3, dst3)
    g2 = _tc_layer2(acc1[0], acc1[1], g1, dega, degb,
                    jnp.reshape(b1, (1, D)), W2)
    acc2 = _sc_aggregate(g2, src3, dst3)

    batf = batch.astype(jnp.int32).reshape(NBLK, 1, ROWBLK)
    wl = jnp.zeros((D, D), jnp.float32).at[:, :N_CLASSES].set(Wlin)
    bl = jnp.zeros((1, D), jnp.float32).at[0, :N_CLASSES].set(blin)
    outp = _tc_final(acc2[0], acc2[1], g2, dega, degb,
                     jnp.reshape(b2, (1, D)), batf, wl, bl)
    return outp[:, :N_CLASSES]


# speedup vs baseline: 13.5338x; 13.5338x over previous
import functools

import jax
import jax.numpy as jnp
from jax import lax
from jax.experimental import pallas as pl
from jax.experimental.pallas import tpu as pltpu
from jax.experimental.pallas import tpu_sc as plsc

N_NODES = 10000
N_EDGES = 320000
D = 128
DH = D // 2
N_GRAPHS = 64
N_CLASSES = 10

NC = 2
NS = 16
CHUNK = 128
CPT = 158
E_PAD = NS * CPT * CHUNK
NPAD = N_NODES + 112
RPT = NPAD // NS
ROWBLK = 1000
NBLK = N_NODES // ROWBLK


def _mesh():
    return plsc.VectorSubcoreMesh(core_axis_name="c", subcore_axis_name="s")


def _zero_rows(ref, nrows, ncols):
    @pl.loop(0, nrows)
    def _(r):
        @pl.loop(0, ncols // 16)
        def _(j):
            ref[r, pl.ds(j * 16, 16)] = jnp.zeros((16,), jnp.float32)


def _sc_degree(dst3):
    @functools.partial(
        pl.kernel,
        out_type=jax.ShapeDtypeStruct((NC, NPAD, 16), jnp.float32),
        mesh=_mesh(),
        compiler_params=pltpu.CompilerParams(use_tc_tiling_on_sc=False),
        scratch_types=[
            pltpu.VMEM((CPT // 2, CHUNK), jnp.int32),
            pltpu.VMEM((CHUNK, 16), jnp.float32),
            pltpu.VMEM((CHUNK, 16), jnp.float32),
            pltpu.VMEM_SHARED((NPAD, 16), jnp.float32),
        ],
    )
    def k(dst_hbm, out_hbm, dst_v, ones_v, zero_v, acc_sh):
        cid = lax.axis_index("c")
        sid = lax.axis_index("s")
        wid = cid * NS + sid

        @pl.loop(0, CHUNK)
        def _(r):
            ones_v[r, pl.ds(0, 16)] = jnp.ones((16,), jnp.float32)
            zero_v[r, pl.ds(0, 16)] = jnp.zeros((16,), jnp.float32)

        @pl.loop(0, 4)
        def _(t):
            pltpu.sync_copy(zero_v, acc_sh.at[pl.ds(sid * RPT + t * CHUNK, CHUNK)])
        pltpu.sync_copy(zero_v.at[pl.ds(0, RPT - 4 * CHUNK)],
                        acc_sh.at[pl.ds(sid * RPT + 4 * CHUNK, RPT - 4 * CHUNK)])
        plsc.subcore_barrier()

        pltpu.sync_copy(dst_hbm.at[wid], dst_v)

        @pl.loop(0, CPT // 2)
        def _(c):
            pltpu.sync_copy(ones_v, acc_sh.at[dst_v.at[c]], add=True)

        plsc.subcore_barrier()
        pltpu.sync_copy(acc_sh.at[pl.ds(sid * RPT, RPT)],
                        out_hbm.at[cid].at[pl.ds(sid * RPT, RPT)])

    return k(dst3)


def _sc_aggregate(gsplit, src3, dst3):
    @functools.partial(
        pl.kernel,
        out_type=jax.ShapeDtypeStruct((NC, NPAD, DH), jnp.float32),
        mesh=_mesh(),
        compiler_params=pltpu.CompilerParams(use_tc_tiling_on_sc=False),
        scratch_types=[
            pltpu.VMEM((CPT, CHUNK), jnp.int32),
            pltpu.VMEM((CPT, CHUNK), jnp.int32),
            pltpu.VMEM((CHUNK, DH), jnp.float32),
            pltpu.VMEM((CHUNK, DH), jnp.float32),
            pltpu.VMEM_SHARED((NPAD, DH), jnp.float32),
            pltpu.SemaphoreType.DMA,
        ],
    )
    def k(g_hbm, src_hbm, dst_hbm, out_hbm, src_v, dst_v, rows_v, zero_v,
          acc_sh, sem):
        cid = lax.axis_index("c")
        sid = lax.axis_index("s")

        _zero_rows(zero_v, CHUNK, DH)

        @pl.loop(0, 4)
        def _(t):
            pltpu.sync_copy(zero_v, acc_sh.at[pl.ds(sid * RPT + t * CHUNK, CHUNK)])
        pltpu.sync_copy(zero_v.at[pl.ds(0, RPT - 4 * CHUNK)],
                        acc_sh.at[pl.ds(sid * RPT + 4 * CHUNK, RPT - 4 * CHUNK)])
        plsc.subcore_barrier()

        pltpu.sync_copy(src_hbm.at[sid], src_v)
        pltpu.sync_copy(dst_hbm.at[sid], dst_v)

        @pl.loop(0, CPT)
        def _(c):
            pltpu.async_copy(g_hbm.at[cid].at[src_v.at[c]], rows_v, sem).wait()
            pltpu.sync_copy(rows_v, acc_sh.at[dst_v.at[c]], add=True)

        plsc.subcore_barrier()
        pltpu.sync_copy(acc_sh.at[pl.ds(sid * RPT, RPT)],
                        out_hbm.at[cid].at[pl.ds(sid * RPT, RPT)])

    return k(gsplit, src3, dst3)


_DOT = (((1,), (0,)), ((), ()))


def _split(h):
    return jnp.stack([h[:, :DH], h[:, DH:]])


def _unsplit(blk):
    return jnp.concatenate([blk[0], blk[1]], axis=-1)


def _tc_layer1(x, dega, degb, W1):
    def body(x_ref, da_ref, db_ref, w_ref, g_ref):
        deg = da_ref[:, 0:1] + db_ref[:, 0:1] + 1.0
        dinv = lax.rsqrt(deg)
        h = lax.dot_general(x_ref[...], w_ref[...], _DOT,
                            precision=lax.Precision.HIGHEST)
        g_ref[...] = _split(h * dinv)

    return pl.pallas_call(
        body,
        grid=(NBLK,),
        in_specs=[
            pl.BlockSpec((ROWBLK, D), lambda i: (i, 0)),
            pl.BlockSpec((ROWBLK, 16), lambda i: (i, 0)),
            pl.BlockSpec((ROWBLK, 16), lambda i: (i, 0)),
            pl.BlockSpec((D, D), lambda i: (0, 0)),
        ],
        out_specs=pl.BlockSpec((NC, ROWBLK, DH), lambda i: (0, i, 0)),
        out_shape=jax.ShapeDtypeStruct((NC, N_NODES, DH), jnp.float32),
    )(x, dega, degb, W1)


def _tc_layer2(acc, g1, dega, degb, b1, W2):
    def body(a_ref, g_ref, da_ref, db_ref, b_ref, w_ref, o_ref):
        deg = da_ref[:, 0:1] + db_ref[:, 0:1] + 1.0
        dinv = lax.rsqrt(deg)
        z = dinv * (_unsplit(a_ref[...]) + _unsplit(g_ref[...])) + b_ref[...]
        h = jnp.maximum(z, 0.0)
        h2 = lax.dot_general(h, w_ref[...], _DOT,
                             precision=lax.Precision.HIGHEST)
        o_ref[...] = _split(h2 * dinv)

    return pl.pallas_call(
        body,
        grid=(NBLK,),
        in_specs=[
            pl.BlockSpec((NC, ROWBLK, DH), lambda i: (0, i, 0)),
            pl.BlockSpec((NC, ROWBLK, DH), lambda i: (0, i, 0)),
            pl.BlockSpec((ROWBLK, 16), lambda i: (i, 0)),
            pl.BlockSpec((ROWBLK, 16), lambda i: (i, 0)),
            pl.BlockSpec((1, D), lambda i: (0, 0)),
            pl.BlockSpec((D, D), lambda i: (0, 0)),
        ],
        out_specs=pl.BlockSpec((NC, ROWBLK, DH), lambda i: (0, i, 0)),
        out_shape=jax.ShapeDtypeStruct((NC, N_NODES, DH), jnp.float32),
    )(acc, g1, dega, degb, b1, W2)


def _tc_final(acc, g2, dega, degb, b2, batf, wl, bl):
    def body(a_ref, g_ref, da_ref, db_ref, b_ref, bat_ref, wl_ref,
             bl_ref, o_ref, pool_ref):
        i = pl.program_id(0)
        deg = da_ref[:, 0:1] + db_ref[:, 0:1] + 1.0
        dinv = lax.rsqrt(deg)
        z = dinv * (_unsplit(a_ref[...]) + _unsplit(g_ref[...])) + b_ref[...]
        h = jnp.maximum(z, 0.0)
        bvec = jnp.reshape(bat_ref[0, 0, :], (1, ROWBLK))
        gids = lax.broadcasted_iota(jnp.int32, (N_GRAPHS, ROWBLK), 0)
        m = (bvec == gids).astype(jnp.float32)
        pm = lax.dot_general(m, h, _DOT, precision=lax.Precision.HIGHEST)

        @pl.when(i == 0)
        def _():
            pool_ref[...] = pm

        @pl.when(i > 0)
        def _():
            pool_ref[...] += pm

        @pl.when(i == NBLK - 1)
        def _():
            o_ref[...] = lax.dot_general(pool_ref[...], wl_ref[...], _DOT,
                                         precision=lax.Precision.HIGHEST) + bl_ref[...]

    return pl.pallas_call(
        body,
        grid=(NBLK,),
        in_specs=[
            pl.BlockSpec((NC, ROWBLK, DH), lambda i: (0, i, 0)),
            pl.BlockSpec((NC, ROWBLK, DH), lambda i: (0, i, 0)),
            pl.BlockSpec((ROWBLK, 16), lambda i: (i, 0)),
            pl.BlockSpec((ROWBLK, 16), lambda i: (i, 0)),
            pl.BlockSpec((1, D), lambda i: (0, 0)),
            pl.BlockSpec((1, 1, ROWBLK), lambda i: (i, 0, 0)),
            pl.BlockSpec((D, D), lambda i: (0, 0)),
            pl.BlockSpec((1, D), lambda i: (0, 0)),
        ],
        out_specs=pl.BlockSpec((N_GRAPHS, D), lambda i: (0, 0)),
        out_shape=jax.ShapeDtypeStruct((N_GRAPHS, D), jnp.float32),
        scratch_shapes=[pltpu.VMEM((N_GRAPHS, D), jnp.float32)],
    )(acc, g2, dega, degb, b2, batf, wl, bl)


def kernel(x, edge_index, batch, W1, b1, W2, b2, Wlin, blin):
    src = edge_index[0].astype(jnp.int32)
    dst = edge_index[1].astype(jnp.int32)
    pad = E_PAD - N_EDGES
    src_p = jnp.concatenate([src, jnp.zeros((pad,), jnp.int32)])
    dst_p = jnp.concatenate([dst, jnp.full((pad,), N_NODES, jnp.int32)])
    src3 = src_p.reshape(NS, CPT, CHUNK)
    dst3 = dst_p.reshape(NS, CPT, CHUNK)
    dst3_32 = dst_p.reshape(NC * NS, CPT // 2, CHUNK)

    deg2 = _sc_degree(dst3_32)
    dega, degb = deg2[0], deg2[1]

    g1 = _tc_layer1(x, dega, degb, W1)
    acc1 = _sc_aggregate(g1, src3, dst3)
    g2 = _tc_layer2(acc1, g1, dega, degb, jnp.reshape(b1, (1, D)), W2)
    acc2 = _sc_aggregate(g2, src3, dst3)

    batf = batch.astype(jnp.int32).reshape(NBLK, 1, ROWBLK)
    wl = jnp.zeros((D, D), jnp.float32).at[:, :N_CLASSES].set(Wlin)
    bl = jnp.zeros((1, D), jnp.float32).at[0, :N_CLASSES].set(blin)
    outp = _tc_final(acc2, g2, dega, degb,
                     jnp.reshape(b2, (1, D)), batf, wl, bl)
    return outp[:, :N_CLASSES]


# speedup vs baseline: 15.7914x; 1.1668x over previous
import functools

import jax
import jax.numpy as jnp
from jax import lax
from jax.experimental import pallas as pl
from jax.experimental.pallas import tpu as pltpu
from jax.experimental.pallas import tpu_sc as plsc

N_NODES = 10000
N_EDGES = 320000
D = 128
DH = D // 2
N_GRAPHS = 64
N_CLASSES = 10

NC = 2
NS = 16
CHUNK = 128
CPT = 158
E_PAD = NS * CPT * CHUNK
NPAD = N_NODES + 112
RPT = NPAD // NS
ROWBLK = 1000
NBLK = N_NODES // ROWBLK


def _mesh():
    return plsc.VectorSubcoreMesh(core_axis_name="c", subcore_axis_name="s")


def _zero_rows(ref, nrows, ncols):
    @pl.loop(0, nrows)
    def _(r):
        @pl.loop(0, ncols // 16)
        def _(j):
            ref[r, pl.ds(j * 16, 16)] = jnp.zeros((16,), jnp.float32)


def _sc_degree(dst3):
    @functools.partial(
        pl.kernel,
        out_type=jax.ShapeDtypeStruct((NC, NPAD, 16), jnp.float32),
        mesh=_mesh(),
        compiler_params=pltpu.CompilerParams(use_tc_tiling_on_sc=False),
        scratch_types=[
            pltpu.VMEM((CPT // 2, CHUNK), jnp.int32),
            pltpu.VMEM((CHUNK, 16), jnp.float32),
            pltpu.VMEM((CHUNK, 16), jnp.float32),
            pltpu.VMEM_SHARED((NPAD, 16), jnp.float32),
        ],
    )
    def k(dst_hbm, out_hbm, dst_v, ones_v, zero_v, acc_sh):
        cid = lax.axis_index("c")
        sid = lax.axis_index("s")
        wid = cid * NS + sid

        @pl.loop(0, CHUNK)
        def _(r):
            ones_v[r, pl.ds(0, 16)] = jnp.ones((16,), jnp.float32)
            zero_v[r, pl.ds(0, 16)] = jnp.zeros((16,), jnp.float32)

        @pl.loop(0, 4)
        def _(t):
            pltpu.sync_copy(zero_v, acc_sh.at[pl.ds(sid * RPT + t * CHUNK, CHUNK)])
        pltpu.sync_copy(zero_v.at[pl.ds(0, RPT - 4 * CHUNK)],
                        acc_sh.at[pl.ds(sid * RPT + 4 * CHUNK, RPT - 4 * CHUNK)])
        plsc.subcore_barrier()

        pltpu.sync_copy(dst_hbm.at[wid], dst_v)

        @pl.loop(0, CPT // 2)
        def _(c):
            pltpu.sync_copy(ones_v, acc_sh.at[dst_v.at[c]], add=True)

        plsc.subcore_barrier()
        pltpu.sync_copy(acc_sh.at[pl.ds(sid * RPT, RPT)],
                        out_hbm.at[cid].at[pl.ds(sid * RPT, RPT)])

    return k(dst3)


def _sc_aggregate(gsplit, src3, dst3):
    @functools.partial(
        pl.kernel,
        out_type=jax.ShapeDtypeStruct((NC, NPAD, DH), jnp.float32),
        mesh=_mesh(),
        compiler_params=pltpu.CompilerParams(use_tc_tiling_on_sc=False),
        scratch_types=[
            pltpu.VMEM((CPT, CHUNK), jnp.int32),
            pltpu.VMEM((CPT, CHUNK), jnp.int32),
            pltpu.VMEM((CHUNK, DH), jnp.float32),
            pltpu.VMEM((CHUNK, DH), jnp.float32),
            pltpu.VMEM((CHUNK, DH), jnp.float32),
            pltpu.VMEM_SHARED((NPAD, DH), jnp.float32),
            pltpu.SemaphoreType.DMA,
            pltpu.SemaphoreType.DMA,
        ],
    )
    def k(g_hbm, src_hbm, dst_hbm, out_hbm, src_v, dst_v, rows0, rows1,
          zero_v, acc_sh, sem0, sem1):
        cid = lax.axis_index("c")
        sid = lax.axis_index("s")

        pltpu.sync_copy(src_hbm.at[sid], src_v)
        pltpu.sync_copy(dst_hbm.at[sid], dst_v)

        _zero_rows(zero_v, CHUNK, DH)

        @pl.loop(0, 4)
        def _(t):
            pltpu.sync_copy(zero_v, acc_sh.at[pl.ds(sid * RPT + t * CHUNK, CHUNK)])
        pltpu.sync_copy(zero_v.at[pl.ds(0, RPT - 4 * CHUNK)],
                        acc_sh.at[pl.ds(sid * RPT + 4 * CHUNK, RPT - 4 * CHUNK)])
        plsc.subcore_barrier()

        gsrc = g_hbm.at[cid]
        pltpu.async_copy(gsrc.at[src_v.at[0]], rows0, sem0)

        @pl.loop(0, CPT // 2)
        def _(p):
            c0 = 2 * p
            pltpu.make_async_copy(gsrc.at[src_v.at[0]], rows0, sem0).wait()
            pltpu.async_copy(gsrc.at[src_v.at[c0 + 1]], rows1, sem1)
            pltpu.sync_copy(rows0, acc_sh.at[dst_v.at[c0]], add=True)
            pltpu.make_async_copy(gsrc.at[src_v.at[0]], rows1, sem1).wait()

            @pl.when(p < CPT // 2 - 1)
            def _():
                pltpu.async_copy(gsrc.at[src_v.at[c0 + 2]], rows0, sem0)

            pltpu.sync_copy(rows1, acc_sh.at[dst_v.at[c0 + 1]], add=True)

        plsc.subcore_barrier()
        pltpu.sync_copy(acc_sh.at[pl.ds(sid * RPT, RPT)],
                        out_hbm.at[cid].at[pl.ds(sid * RPT, RPT)])

    return k(gsplit, src3, dst3)


_DOT = (((1,), (0,)), ((), ()))


def _split(h):
    return jnp.stack([h[:, :DH], h[:, DH:]])


def _unsplit(blk):
    return jnp.concatenate([blk[0], blk[1]], axis=-1)


def _tc_layer1(x, dega, degb, W1):
    def body(x_ref, da_ref, db_ref, w_ref, g_ref):
        deg = da_ref[:, 0:1] + db_ref[:, 0:1] + 1.0
        dinv = lax.rsqrt(deg)
        h = lax.dot_general(x_ref[...], w_ref[...], _DOT,
                            precision=lax.Precision.HIGHEST)
        g_ref[...] = _split(h * dinv)

    return pl.pallas_call(
        body,
        grid=(NBLK,),
        in_specs=[
            pl.BlockSpec((ROWBLK, D), lambda i: (i, 0)),
            pl.BlockSpec((ROWBLK, 16), lambda i: (i, 0)),
            pl.BlockSpec((ROWBLK, 16), lambda i: (i, 0)),
            pl.BlockSpec((D, D), lambda i: (0, 0)),
        ],
        out_specs=pl.BlockSpec((NC, ROWBLK, DH), lambda i: (0, i, 0)),
        out_shape=jax.ShapeDtypeStruct((NC, N_NODES, DH), jnp.float32),
    )(x, dega, degb, W1)


def _tc_layer2(acc, g1, dega, degb, b1, W2):
    def body(a_ref, g_ref, da_ref, db_ref, b_ref, w_ref, o_ref):
        deg = da_ref[:, 0:1] + db_ref[:, 0:1] + 1.0
        dinv = lax.rsqrt(deg)
        z = dinv * (_unsplit(a_ref[...]) + _unsplit(g_ref[...])) + b_ref[...]
        h = jnp.maximum(z, 0.0)
        h2 = lax.dot_general(h, w_ref[...], _DOT,
                             precision=lax.Precision.HIGHEST)
        o_ref[...] = _split(h2 * dinv)

    return pl.pallas_call(
        body,
        grid=(NBLK,),
        in_specs=[
            pl.BlockSpec((NC, ROWBLK, DH), lambda i: (0, i, 0)),
            pl.BlockSpec((NC, ROWBLK, DH), lambda i: (0, i, 0)),
            pl.BlockSpec((ROWBLK, 16), lambda i: (i, 0)),
            pl.BlockSpec((ROWBLK, 16), lambda i: (i, 0)),
            pl.BlockSpec((1, D), lambda i: (0, 0)),
            pl.BlockSpec((D, D), lambda i: (0, 0)),
        ],
        out_specs=pl.BlockSpec((NC, ROWBLK, DH), lambda i: (0, i, 0)),
        out_shape=jax.ShapeDtypeStruct((NC, N_NODES, DH), jnp.float32),
    )(acc, g1, dega, degb, b1, W2)


def _tc_final(acc, g2, dega, degb, b2, batf, wl, bl):
    def body(a_ref, g_ref, da_ref, db_ref, b_ref, bat_ref, wl_ref,
             bl_ref, o_ref, pool_ref):
        i = pl.program_id(0)
        deg = da_ref[:, 0:1] + db_ref[:, 0:1] + 1.0
        dinv = lax.rsqrt(deg)
        z = dinv * (_unsplit(a_ref[...]) + _unsplit(g_ref[...])) + b_ref[...]
        h = jnp.maximum(z, 0.0)
        bvec = jnp.reshape(bat_ref[0, 0, :], (1, ROWBLK))
        gids = lax.broadcasted_iota(jnp.int32, (N_GRAPHS, ROWBLK), 0)
        m = (bvec == gids).astype(jnp.float32)
        pm = lax.dot_general(m, h, _DOT, precision=lax.Precision.HIGHEST)

        @pl.when(i == 0)
        def _():
            pool_ref[...] = pm

        @pl.when(i > 0)
        def _():
            pool_ref[...] += pm

        @pl.when(i == NBLK - 1)
        def _():
            o_ref[...] = lax.dot_general(pool_ref[...], wl_ref[...], _DOT,
                                         precision=lax.Precision.HIGHEST) + bl_ref[...]

    return pl.pallas_call(
        body,
        grid=(NBLK,),
        in_specs=[
            pl.BlockSpec((NC, ROWBLK, DH), lambda i: (0, i, 0)),
            pl.BlockSpec((NC, ROWBLK, DH), lambda i: (0, i, 0)),
            pl.BlockSpec((ROWBLK, 16), lambda i: (i, 0)),
            pl.BlockSpec((ROWBLK, 16), lambda i: (i, 0)),
            pl.BlockSpec((1, D), lambda i: (0, 0)),
            pl.BlockSpec((1, 1, ROWBLK), lambda i: (i, 0, 0)),
            pl.BlockSpec((D, D), lambda i: (0, 0)),
            pl.BlockSpec((1, D), lambda i: (0, 0)),
        ],
        out_specs=pl.BlockSpec((N_GRAPHS, D), lambda i: (0, 0)),
        out_shape=jax.ShapeDtypeStruct((N_GRAPHS, D), jnp.float32),
        scratch_shapes=[pltpu.VMEM((N_GRAPHS, D), jnp.float32)],
    )(acc, g2, dega, degb, b2, batf, wl, bl)


def kernel(x, edge_index, batch, W1, b1, W2, b2, Wlin, blin):
    src = edge_index[0].astype(jnp.int32)
    dst = edge_index[1].astype(jnp.int32)
    pad = E_PAD - N_EDGES
    src_p = jnp.concatenate([src, jnp.zeros((pad,), jnp.int32)])
    dst_p = jnp.concatenate([dst, jnp.full((pad,), N_NODES, jnp.int32)])
    src3 = src_p.reshape(NS, CPT, CHUNK)
    dst3 = dst_p.reshape(NS, CPT, CHUNK)
    dst3_32 = dst_p.reshape(NC * NS, CPT // 2, CHUNK)

    deg2 = _sc_degree(dst3_32)
    dega, degb = deg2[0], deg2[1]

    g1 = _tc_layer1(x, dega, degb, W1)
    acc1 = _sc_aggregate(g1, src3, dst3)
    g2 = _tc_layer2(acc1, g1, dega, degb, jnp.reshape(b1, (1, D)), W2)
    acc2 = _sc_aggregate(g2, src3, dst3)

    batf = batch.astype(jnp.int32).reshape(NBLK, 1, ROWBLK)
    wl = jnp.zeros((D, D), jnp.float32).at[:, :N_CLASSES].set(Wlin)
    bl = jnp.zeros((1, D), jnp.float32).at[0, :N_CLASSES].set(blin)
    outp = _tc_final(acc2, g2, dega, degb,
                     jnp.reshape(b2, (1, D)), batf, wl, bl)
    return outp[:, :N_CLASSES]
